# baseline (speedup 1.0000x reference)
import functools

import jax
import jax.numpy as jnp
from jax import lax
from jax.experimental import pallas as pl
from jax.experimental.pallas import tpu as pltpu
from jax.experimental.pallas import tpu_sc as plsc

_NUM_OBJS = 1000
_NUM_RELS = 64
_BATCH = 16384
_L = 16
_IDX_CHUNK = 128


@functools.lru_cache(maxsize=None)
def _build(num_cores: int, num_subcores: int):
    nw = num_cores * num_subcores
    bpw = _BATCH // nw
    n_chunks = bpw // _IDX_CHUNK
    mesh = plsc.VectorSubcoreMesh(
        core_axis_name="c", subcore_axis_name="s",
        num_cores=num_cores, num_subcores=num_subcores)

    @functools.partial(
        pl.kernel,
        out_type=jax.ShapeDtypeStruct((_BATCH, _NUM_RELS), jnp.float32),
        mesh=mesh,
        scratch_types=[
            pltpu.VMEM((2 * bpw,), jnp.int32),
            pltpu.VMEM((bpw, _NUM_RELS), jnp.float32),
            pltpu.SemaphoreType.DMA,
        ],
        compiler_params=pltpu.CompilerParams(use_tc_tiling_on_sc=False),
    )
    def k(labels_hbm, table_hbm, out_hbm, lab_v, rows_v, sem):
        wid = lax.axis_index("s") * num_cores + lax.axis_index("c")
        base = wid * bpw
        pltpu.sync_copy(labels_hbm.at[pl.ds(base * 2, 2 * bpw)], lab_v)
        lane = lax.iota(jnp.int32, _L)
        half = lane < 8
        even2 = (lane & 7) * 2
        odd2 = even2 + 1
        copies = []
        for j in range(bpw // _L):
            a = lab_v[pl.ds(2 * _L * j, _L)]
            b = lab_v[pl.ds(2 * _L * j + _L, _L)]
            l0 = jnp.where(half,
                           a.at[even2].get(mode="promise_in_bounds"),
                           b.at[even2].get(mode="promise_in_bounds"))
            l1 = jnp.where(half,
                           a.at[odd2].get(mode="promise_in_bounds"),
                           b.at[odd2].get(mode="promise_in_bounds"))
            fused = l0 * _NUM_OBJS + l1
            copies.append(pltpu.async_copy(
                table_hbm.at[fused],
                rows_v.at[pl.ds(j * _L, _L)],
                sem))
        for cp in copies:
            cp.wait()
        pltpu.sync_copy(rows_v, out_hbm.at[pl.ds(base, bpw)])

    return k


def kernel(labels, obj_baseline):
    info = plsc.get_sparse_core_info()
    k = _build(info.num_cores, info.num_subcores)
    return k(labels.reshape(-1), obj_baseline)
